# baseline (speedup 1.0000x reference)
import functools

import jax
import jax.numpy as jnp
from jax import lax
from jax.experimental import pallas as pl
from jax.experimental.pallas import tpu as pltpu
from jax.experimental.pallas import tpu_sc as plsc

POS_IOU = 0.5
NEG_IOU = 0.4
LOW_IOU = 0.1
EPS = 1e-6

B, N, M = 8, 20000, 128
BN = 5120
N_PAD = 20480
NB = N_PAD // BN


def _tc_body(num_s, bt_ref, g_ref, fm_ref, winf_ref, winm_ref, vals_ref,
             gmax_s, garg_s):
    b = pl.program_id(0)
    t = pl.program_id(1)

    bt = bt_ref[0]
    bx1 = bt[0:1, :]
    by1 = bt[1:2, :]
    bx2 = bt[2:3, :]
    by2 = bt[3:4, :]
    g = g_ref[0]
    gx1 = g[:, 0:1]
    gy1 = g[:, 1:2]
    gx2 = g[:, 2:3]
    gy2 = g[:, 3:4]

    ix1 = jnp.maximum(bx1, gx1)
    iy1 = jnp.maximum(by1, gy1)
    ix2 = jnp.minimum(bx2, gx2)
    iy2 = jnp.minimum(by2, gy2)
    iw = jnp.maximum(ix2 - ix1, 0.0)
    ih = jnp.maximum(iy2 - iy1, 0.0)
    inter = iw * ih
    area_b = (bx2 - bx1) * (by2 - by1)
    area_g = (gx2 - gx1) * (gy2 - gy1)
    denom = jnp.maximum(area_b + area_g - inter, EPS)
    iou = inter / denom

    num = num_s[b]
    gio = lax.broadcasted_iota(jnp.int32, (M, 1), 0)
    vmask = gio < num
    iou = jnp.where(vmask, iou, -1.0)

    amax = jnp.max(iou, axis=0, keepdims=True)
    sio = lax.broadcasted_iota(jnp.int32, (M, BN), 0)
    aarg = jnp.min(jnp.where(iou == amax, sio, M), axis=0, keepdims=True)
    flag = jnp.where(amax < NEG_IOU, jnp.int32(0), jnp.int32(-1))
    flag = jnp.where(amax >= POS_IOU, jnp.int32(1), flag)
    match = jnp.where(flag == 1, aarg, jnp.int32(-1))
    fm_ref[0, 0:1, pl.ds(t * BN, BN)] = flag
    fm_ref[0, 1:2, pl.ds(t * BN, BN)] = match

    gmax_t = jnp.max(iou, axis=1, keepdims=True)
    lio = lax.broadcasted_iota(jnp.int32, (M, BN), 1) + t * BN
    garg_t = jnp.min(jnp.where(iou == gmax_t, lio, N_PAD), axis=1, keepdims=True)

    @pl.when(t == 0)
    def _():
        gmax_s[...] = gmax_t
        garg_s[...] = garg_t

    @pl.when(t > 0)
    def _():
        prev_max = gmax_s[...]
        prev_arg = garg_s[...]
        better = gmax_t > prev_max
        gmax_s[...] = jnp.where(better, gmax_t, prev_max)
        garg_s[...] = jnp.where(better, garg_t, prev_arg)

    @pl.when(t == NB - 1)
    def _():
        valid = vmask & (gmax_s[...] >= LOW_IOU)
        win_col = jnp.where(valid, garg_s[...], jnp.int32(N))
        win_row = jnp.transpose(win_col)
        eq = win_col == win_row
        gp = lax.broadcasted_iota(jnp.int32, (M, M), 1)
        resolved = jnp.max(jnp.where(eq, gp, -1), axis=1, keepdims=True)
        winf_ref[0] = win_col + b * (2 * N_PAD)
        winm_ref[0] = win_col + b * (2 * N_PAD) + N_PAD
        vals_ref[0] = resolved


def _sc_body(bh, winf_h, winm_h, vals_h, fm_h, fm_o, winf_v, winm_v, vals_v,
             ones_v, row_v, sem):
    c = lax.axis_index("c")
    s = lax.axis_index("s")

    ch = bh * 2 * N_PAD // 32
    base = c * (bh // 2) * (2 * N_PAD) + s * ch
    pltpu.sync_copy(fm_h.at[pl.ds(base, ch)], row_v)
    pltpu.sync_copy(row_v, fm_o.at[pl.ds(base, ch)])

    plsc.subcore_barrier()

    @pl.when(s < bh // 2)
    def _():
        b = c * (bh // 2) + s
        pltpu.sync_copy(winf_h.at[b], winf_v)
        pltpu.sync_copy(winm_h.at[b], winm_v)
        pltpu.sync_copy(vals_h.at[b], vals_v)
        for j in range(M // 16):
            ones_v[pl.ds(j * 16, 16)] = jnp.full((16,), 1, jnp.int32)
        pltpu.async_copy(ones_v, fm_o.at[winf_v], sem).wait()
        pltpu.async_copy(vals_v, fm_o.at[winm_v], sem).wait()


def _tc_call(boxes_t, g4, num, bh):
    return pl.pallas_call(
        _tc_body,
        grid=(bh, NB),
        in_specs=[
            pl.BlockSpec(memory_space=pltpu.SMEM),
            pl.BlockSpec((1, 8, BN), lambda b, t: (b, 0, t)),
            pl.BlockSpec((1, M, 4), lambda b, t: (b, 0, 0)),
        ],
        out_specs=[
            pl.BlockSpec((1, 2, N_PAD), lambda b, t: (b, 0, 0)),
            pl.BlockSpec((1, M, 1), lambda b, t: (b, 0, 0)),
            pl.BlockSpec((1, M, 1), lambda b, t: (b, 0, 0)),
            pl.BlockSpec((1, M, 1), lambda b, t: (b, 0, 0)),
        ],
        out_shape=[
            jax.ShapeDtypeStruct((bh, 2, N_PAD), jnp.int32),
            jax.ShapeDtypeStruct((bh, M, 1), jnp.int32),
            jax.ShapeDtypeStruct((bh, M, 1), jnp.int32),
            jax.ShapeDtypeStruct((bh, M, 1), jnp.int32),
        ],
        scratch_shapes=[
            pltpu.VMEM((M, 1), jnp.float32),
            pltpu.VMEM((M, 1), jnp.int32),
        ],
        compiler_params=pltpu.CompilerParams(
            dimension_semantics=("arbitrary", "arbitrary")),
    )(num, boxes_t, g4)


@functools.cache
def _sc_apply(bh):
    return pl.kernel(
        functools.partial(_sc_body, bh),
        out_type=jax.ShapeDtypeStruct((bh * 2 * N_PAD,), jnp.int32),
        mesh=plsc.VectorSubcoreMesh(core_axis_name="c", subcore_axis_name="s"),
        scratch_types=[
            pltpu.VMEM((M,), jnp.int32),
            pltpu.VMEM((M,), jnp.int32),
            pltpu.VMEM((M,), jnp.int32),
            pltpu.VMEM((M,), jnp.int32),
            pltpu.VMEM((bh * 2 * N_PAD // 32,), jnp.int32),
            pltpu.SemaphoreType.DMA,
        ],
    )


def _half(boxes_t, g4, num, bh):
    fm0, winf, winm, vals = _tc_call(boxes_t, g4, num, bh)
    fm1 = _sc_apply(bh)(
        winf[:, :, 0], winm[:, :, 0], vals[:, :, 0],
        fm0.reshape(bh * 2 * N_PAD))
    fm1 = fm1.reshape(bh, 2, N_PAD)
    return fm1[:, 0, :N], fm1[:, 1, :N]


def kernel(boxes, gt_boxes, gt_boxes_num):
    boxes_t = jnp.transpose(boxes, (0, 2, 1))
    boxes_t = jnp.pad(boxes_t, ((0, 0), (0, 4), (0, N_PAD - N)))
    g4 = gt_boxes[:, :, :4]
    num = gt_boxes_num.astype(jnp.int32)

    return _half(boxes_t, g4, num, B)


# speedup vs baseline: 1.0821x; 1.0821x over previous
import functools

import jax
import jax.numpy as jnp
from jax import lax
from jax.experimental import pallas as pl
from jax.experimental.pallas import tpu as pltpu
from jax.experimental.pallas import tpu_sc as plsc

POS_IOU = 0.5
NEG_IOU = 0.4
LOW_IOU = 0.1
EPS = 1e-6

B, N, M = 8, 20000, 128
BN = 5120
N_PAD = 20480
NB = N_PAD // BN


def _tc_body(num_s, bt_ref, g_ref, flag_ref, match_ref, win_ref, vals_ref,
             gmax_s, garg_s):
    b = pl.program_id(0)
    t = pl.program_id(1)

    bt = bt_ref[0]
    bx1 = bt[0:1, :]
    by1 = bt[1:2, :]
    bx2 = bt[2:3, :]
    by2 = bt[3:4, :]
    g = g_ref[0]
    gx1 = g[:, 0:1]
    gy1 = g[:, 1:2]
    gx2 = g[:, 2:3]
    gy2 = g[:, 3:4]

    ix1 = jnp.maximum(bx1, gx1)
    iy1 = jnp.maximum(by1, gy1)
    ix2 = jnp.minimum(bx2, gx2)
    iy2 = jnp.minimum(by2, gy2)
    iw = jnp.maximum(ix2 - ix1, 0.0)
    ih = jnp.maximum(iy2 - iy1, 0.0)
    inter = iw * ih
    area_b = (bx2 - bx1) * (by2 - by1)
    area_g = (gx2 - gx1) * (gy2 - gy1)
    denom = jnp.maximum(area_b + area_g - inter, EPS)
    iou = inter / denom

    num = num_s[b]
    gio = lax.broadcasted_iota(jnp.int32, (M, 1), 0)
    vmask = gio < num
    iou = jnp.where(vmask, iou, -1.0)

    amax = jnp.max(iou, axis=0, keepdims=True)
    sio = lax.broadcasted_iota(jnp.int32, (M, BN), 0)
    aarg = jnp.min(jnp.where(iou == amax, sio, M), axis=0, keepdims=True)
    flag = jnp.where(amax < NEG_IOU, jnp.int32(0), jnp.int32(-1))
    flag = jnp.where(amax >= POS_IOU, jnp.int32(1), flag)
    match = jnp.where(flag == 1, aarg, jnp.int32(-1))
    flag_ref[0, 0:1, pl.ds(t * BN, BN)] = flag
    match_ref[0, 0:1, pl.ds(t * BN, BN)] = match

    gmax_t = jnp.max(iou, axis=1, keepdims=True)
    lio = lax.broadcasted_iota(jnp.int32, (M, BN), 1) + t * BN
    garg_t = jnp.min(jnp.where(iou == gmax_t, lio, N_PAD), axis=1, keepdims=True)

    @pl.when(t == 0)
    def _():
        gmax_s[...] = gmax_t
        garg_s[...] = garg_t

    @pl.when(t > 0)
    def _():
        prev_max = gmax_s[...]
        prev_arg = garg_s[...]
        better = gmax_t > prev_max
        gmax_s[...] = jnp.where(better, gmax_t, prev_max)
        garg_s[...] = jnp.where(better, garg_t, prev_arg)

    @pl.when(t == NB - 1)
    def _():
        valid = vmask & (gmax_s[...] >= LOW_IOU)
        win_col = jnp.where(valid, garg_s[...], jnp.int32(N))
        win_row = jnp.transpose(win_col)
        eq = win_col == win_row
        gp = lax.broadcasted_iota(jnp.int32, (M, M), 1)
        resolved = jnp.max(jnp.where(eq, gp, -1), axis=1, keepdims=True)
        win_ref[0] = win_col + b * N_PAD
        vals_ref[0] = resolved


def _sc_body(bh, win_h, vals_h, flag_h, match_h, flag_o, match_o, win_v,
             vals_v, row_v, sem):
    c = lax.axis_index("c")
    s = lax.axis_index("s")

    ch = N_PAD * bh // 16
    chunk_base = s * ch

    @pl.when(c == 0)
    def _():
        pltpu.sync_copy(flag_h.at[pl.ds(chunk_base, ch)], row_v)
        pltpu.sync_copy(row_v, flag_o.at[pl.ds(chunk_base, ch)])

    @pl.when(c == 1)
    def _():
        pltpu.sync_copy(match_h.at[pl.ds(chunk_base, ch)], row_v)
        pltpu.sync_copy(row_v, match_o.at[pl.ds(chunk_base, ch)])

    plsc.subcore_barrier()

    @pl.when(s < bh)
    def _():
        b = s
        pltpu.sync_copy(win_h.at[b], win_v)

        @pl.when(c == 0)
        def _():
            for j in range(M // 16):
                vals_v[pl.ds(j * 16, 16)] = jnp.full((16,), 1, jnp.int32)
            pltpu.async_copy(vals_v, flag_o.at[win_v], sem).wait()

        @pl.when(c == 1)
        def _():
            pltpu.sync_copy(vals_h.at[b], vals_v)
            pltpu.async_copy(vals_v, match_o.at[win_v], sem).wait()


def _tc_call(boxes_t, g4, num, bh):
    return pl.pallas_call(
        _tc_body,
        grid=(bh, NB),
        in_specs=[
            pl.BlockSpec(memory_space=pltpu.SMEM),
            pl.BlockSpec((1, 8, BN), lambda b, t: (b, 0, t)),
            pl.BlockSpec((1, M, 4), lambda b, t: (b, 0, 0)),
        ],
        out_specs=[
            pl.BlockSpec((1, 1, N_PAD), lambda b, t: (b, 0, 0)),
            pl.BlockSpec((1, 1, N_PAD), lambda b, t: (b, 0, 0)),
            pl.BlockSpec((1, M, 1), lambda b, t: (b, 0, 0)),
            pl.BlockSpec((1, M, 1), lambda b, t: (b, 0, 0)),
        ],
        out_shape=[
            jax.ShapeDtypeStruct((bh, 1, N_PAD), jnp.int32),
            jax.ShapeDtypeStruct((bh, 1, N_PAD), jnp.int32),
            jax.ShapeDtypeStruct((bh, M, 1), jnp.int32),
            jax.ShapeDtypeStruct((bh, M, 1), jnp.int32),
        ],
        scratch_shapes=[
            pltpu.VMEM((M, 1), jnp.float32),
            pltpu.VMEM((M, 1), jnp.int32),
        ],
        compiler_params=pltpu.CompilerParams(
            dimension_semantics=("arbitrary", "arbitrary")),
    )(num, boxes_t, g4)


@functools.cache
def _sc_apply(bh):
    return pl.kernel(
        functools.partial(_sc_body, bh),
        out_type=(
            jax.ShapeDtypeStruct((bh * N_PAD,), jnp.int32),
            jax.ShapeDtypeStruct((bh * N_PAD,), jnp.int32),
        ),
        mesh=plsc.VectorSubcoreMesh(core_axis_name="c", subcore_axis_name="s"),
        scratch_types=[
            pltpu.VMEM((M,), jnp.int32),
            pltpu.VMEM((M,), jnp.int32),
            pltpu.VMEM((N_PAD * bh // 16,), jnp.int32),
            pltpu.SemaphoreType.DMA,
        ],
    )


def _half(boxes_t, g4, num, bh):
    flag0, match0, win, vals = _tc_call(boxes_t, g4, num, bh)
    flag1, match1 = _sc_apply(bh)(
        win[:, :, 0], vals[:, :, 0],
        flag0.reshape(bh * N_PAD), match0.reshape(bh * N_PAD))
    return (flag1.reshape(bh, N_PAD)[:, :N],
            match1.reshape(bh, N_PAD)[:, :N])


def kernel(boxes, gt_boxes, gt_boxes_num):
    boxes_t = jnp.transpose(boxes, (0, 2, 1))
    boxes_t = jnp.pad(boxes_t, ((0, 0), (0, 4), (0, N_PAD - N)))
    g4 = gt_boxes[:, :, :4]
    num = gt_boxes_num.astype(jnp.int32)

    f, m = _half(boxes_t, g4, num, B)
    return f, m


# speedup vs baseline: 1.1032x; 1.0194x over previous
import functools

import jax
import jax.numpy as jnp
from jax import lax
from jax.experimental import pallas as pl
from jax.experimental.pallas import tpu as pltpu
from jax.experimental.pallas import tpu_sc as plsc

POS_IOU = 0.5
NEG_IOU = 0.4
LOW_IOU = 0.1
EPS = 1e-6

B, N, M = 8, 20000, 128
BN = 5120
N_PAD = 20480
NB = N_PAD // BN


def _tc_body(num_s, bt_ref, g_ref, flag_ref, match_ref, win_ref, vals_ref,
             gmax_s, garg_s):
    b = pl.program_id(0)
    t = pl.program_id(1)

    bt = bt_ref[0]
    bx1 = bt[0:1, :]
    by1 = bt[1:2, :]
    bx2 = bt[2:3, :]
    by2 = bt[3:4, :]
    g = g_ref[0]
    gx1 = g[:, 0:1]
    gy1 = g[:, 1:2]
    gx2 = g[:, 2:3]
    gy2 = g[:, 3:4]

    ix1 = jnp.maximum(bx1, gx1)
    iy1 = jnp.maximum(by1, gy1)
    ix2 = jnp.minimum(bx2, gx2)
    iy2 = jnp.minimum(by2, gy2)
    iw = jnp.maximum(ix2 - ix1, 0.0)
    ih = jnp.maximum(iy2 - iy1, 0.0)
    inter = iw * ih
    area_b = (bx2 - bx1) * (by2 - by1)
    area_g = (gx2 - gx1) * (gy2 - gy1)
    denom = jnp.maximum(area_b + area_g - inter, EPS)
    iou = inter / denom

    amax = jnp.max(iou, axis=0, keepdims=True)
    sio = lax.broadcasted_iota(jnp.int32, (M, BN), 0)
    aarg = jnp.min(jnp.where(iou == amax, sio, M), axis=0, keepdims=True)
    flag = jnp.where(amax < NEG_IOU, jnp.int32(0), jnp.int32(-1))
    flag = jnp.where(amax >= POS_IOU, jnp.int32(1), flag)
    match = jnp.where(flag == 1, aarg, jnp.int32(-1))
    flag_ref[0, 0:1, pl.ds(t * BN, BN)] = flag
    match_ref[0, 0:1, pl.ds(t * BN, BN)] = match

    gmax_t = jnp.max(iou, axis=1, keepdims=True)
    lio = lax.broadcasted_iota(jnp.int32, (M, BN), 1)
    garg_t = jnp.min(jnp.where(iou == gmax_t, lio, BN),
                     axis=1, keepdims=True) + t * BN

    @pl.when(t == 0)
    def _():
        gmax_s[...] = gmax_t
        garg_s[...] = garg_t

    @pl.when(t > 0)
    def _():
        prev_max = gmax_s[...]
        prev_arg = garg_s[...]
        better = gmax_t > prev_max
        gmax_s[...] = jnp.where(better, gmax_t, prev_max)
        garg_s[...] = jnp.where(better, garg_t, prev_arg)

    @pl.when(t == NB - 1)
    def _():
        num = num_s[b]
        gio = lax.broadcasted_iota(jnp.int32, (M, 1), 0)
        vmask = gio < num
        valid = vmask & (gmax_s[...] >= LOW_IOU)
        win_col = jnp.where(valid, garg_s[...], jnp.int32(N))
        win_row = jnp.transpose(win_col)
        eq = win_col == win_row
        gp = lax.broadcasted_iota(jnp.int32, (M, M), 1)
        resolved = jnp.max(jnp.where(eq, gp, -1), axis=1, keepdims=True)
        win_ref[0] = win_col + b * N_PAD
        vals_ref[0] = resolved


def _sc_body(bh, win_h, vals_h, flag_h, match_h, flag_o, match_o, win_v,
             vals_v, row_v, sem):
    c = lax.axis_index("c")
    s = lax.axis_index("s")

    ch = N_PAD * bh // 16
    chunk_base = s * ch

    @pl.when(c == 0)
    def _():
        pltpu.sync_copy(flag_h.at[pl.ds(chunk_base, ch)], row_v)
        pltpu.sync_copy(row_v, flag_o.at[pl.ds(chunk_base, ch)])

    @pl.when(c == 1)
    def _():
        pltpu.sync_copy(match_h.at[pl.ds(chunk_base, ch)], row_v)
        pltpu.sync_copy(row_v, match_o.at[pl.ds(chunk_base, ch)])

    plsc.subcore_barrier()

    @pl.when(s < bh)
    def _():
        b = s
        pltpu.sync_copy(win_h.at[b], win_v)

        @pl.when(c == 0)
        def _():
            for j in range(M // 16):
                vals_v[pl.ds(j * 16, 16)] = jnp.full((16,), 1, jnp.int32)
            pltpu.async_copy(vals_v, flag_o.at[win_v], sem).wait()

        @pl.when(c == 1)
        def _():
            pltpu.sync_copy(vals_h.at[b], vals_v)
            pltpu.async_copy(vals_v, match_o.at[win_v], sem).wait()


def _tc_call(boxes_t, g4, num, bh):
    return pl.pallas_call(
        _tc_body,
        grid=(bh, NB),
        in_specs=[
            pl.BlockSpec(memory_space=pltpu.SMEM),
            pl.BlockSpec((1, 8, BN), lambda b, t: (b, 0, t)),
            pl.BlockSpec((1, M, 4), lambda b, t: (b, 0, 0)),
        ],
        out_specs=[
            pl.BlockSpec((1, 1, N_PAD), lambda b, t: (b, 0, 0)),
            pl.BlockSpec((1, 1, N_PAD), lambda b, t: (b, 0, 0)),
            pl.BlockSpec((1, M, 1), lambda b, t: (b, 0, 0)),
            pl.BlockSpec((1, M, 1), lambda b, t: (b, 0, 0)),
        ],
        out_shape=[
            jax.ShapeDtypeStruct((bh, 1, N_PAD), jnp.int32),
            jax.ShapeDtypeStruct((bh, 1, N_PAD), jnp.int32),
            jax.ShapeDtypeStruct((bh, M, 1), jnp.int32),
            jax.ShapeDtypeStruct((bh, M, 1), jnp.int32),
        ],
        scratch_shapes=[
            pltpu.VMEM((M, 1), jnp.float32),
            pltpu.VMEM((M, 1), jnp.int32),
        ],
        compiler_params=pltpu.CompilerParams(
            dimension_semantics=("arbitrary", "arbitrary")),
    )(num, boxes_t, g4)


@functools.cache
def _sc_apply(bh):
    return pl.kernel(
        functools.partial(_sc_body, bh),
        out_type=(
            jax.ShapeDtypeStruct((bh * N_PAD,), jnp.int32),
            jax.ShapeDtypeStruct((bh * N_PAD,), jnp.int32),
        ),
        mesh=plsc.VectorSubcoreMesh(core_axis_name="c", subcore_axis_name="s"),
        scratch_types=[
            pltpu.VMEM((M,), jnp.int32),
            pltpu.VMEM((M,), jnp.int32),
            pltpu.VMEM((N_PAD * bh // 16,), jnp.int32),
            pltpu.SemaphoreType.DMA,
        ],
    )


def _half(boxes_t, g4, num, bh):
    flag0, match0, win, vals = _tc_call(boxes_t, g4, num, bh)
    flag1, match1 = _sc_apply(bh)(
        win[:, :, 0], vals[:, :, 0],
        flag0.reshape(bh * N_PAD), match0.reshape(bh * N_PAD))
    return (flag1.reshape(bh, N_PAD)[:, :N],
            match1.reshape(bh, N_PAD)[:, :N])


def kernel(boxes, gt_boxes, gt_boxes_num):
    boxes_t = jnp.transpose(boxes, (0, 2, 1))
    boxes_t = jnp.pad(boxes_t, ((0, 0), (0, 4), (0, N_PAD - N)))
    num = gt_boxes_num.astype(jnp.int32)
    gvalid = jnp.arange(M, dtype=jnp.int32)[None, :, None] < num[:, None, None]
    g4 = jnp.where(gvalid, gt_boxes[:, :, :4], 0.0)

    f, m = _half(boxes_t, g4, num, B)
    return f, m
